# speedup vs baseline: 4.7395x; 4.7395x over previous
import jax
import jax.numpy as jnp
from jax.experimental import pallas as pl


def kernel(input_ids, table):
    batch_size, seq_len = input_ids.shape
    max_len, d_model = table.shape
    blk = 256

    def body(t_ref, o_ref):
        o_ref[...] = jnp.broadcast_to(t_ref[...][None, :, :], o_ref.shape)

    out = pl.pallas_call(
        body,
        grid=(seq_len // blk,),
        in_specs=[pl.BlockSpec((blk, d_model), lambda i: (i, 0))],
        out_specs=pl.BlockSpec((batch_size, blk, d_model), lambda i: (0, i, 0)),
        out_shape=jax.ShapeDtypeStruct((batch_size, seq_len, d_model), table.dtype),
    )(table)
    return out
